# Initial kernel scaffold; baseline (speedup 1.0000x reference)
import jax
import jax.numpy as jnp
from jax.experimental import pallas as pl


def kernel(init_h, edge_index, edge_weight, W0, b0, W1, b1, W2, b2):
    raise NotImplementedError("write your pallas kernel here")


# speedup vs baseline: 2.9096x; 2.9096x over previous
import functools

import jax
import jax.numpy as jnp
from jax import lax
from jax.experimental import pallas as pl
from jax.experimental.pallas import tpu as pltpu
from jax.experimental.pallas import tpu_sc as plsc

N = 10000
E = 320000
D = 128
NC = 2
NS = 16
LANES = 16

CHUNK = 128
NCHUNK = 80
EDGES_PER_TILE = NCHUNK * CHUNK
E_PAD = EDGES_PER_TILE * NC * NS
ROWS_PER_TILE = 624
ROWS_TAIL = N - NS * ROWS_PER_TILE


def _sc_agg_body(support_hbm, src_hbm, dst_hbm, w_hbm, zeros_hbm, out_hbm,
                 src_v, dst_v, w_v, rows_v, acc_sh, sem):
    c = lax.axis_index("c")
    s = lax.axis_index("s")

    pltpu.sync_copy(src_hbm.at[c, s], src_v)
    pltpu.sync_copy(dst_hbm.at[c, s], dst_v)
    pltpu.sync_copy(w_hbm.at[c, s], w_v)

    row0 = pl.multiple_of(s * ROWS_PER_TILE, 8)
    pltpu.sync_copy(zeros_hbm.at[pl.ds(row0, ROWS_PER_TILE)],
                    acc_sh.at[pl.ds(row0, ROWS_PER_TILE)])

    @pl.when(s == NS - 1)
    def _():
        pltpu.sync_copy(zeros_hbm.at[pl.ds(NS * ROWS_PER_TILE, ROWS_TAIL)],
                        acc_sh.at[pl.ds(NS * ROWS_PER_TILE, ROWS_TAIL)])
    plsc.subcore_barrier()

    def chunk_body(j, carry):
        pltpu.async_copy(support_hbm.at[src_v.at[j]], rows_v, sem).wait()

        def grp_body(g, carry2):
            base = g * LANES
            wvec = w_v[j, pl.ds(base, LANES)]
            for i in range(LANES):
                w = wvec[i]
                for q in range(D // LANES):
                    sl = pl.ds(q * LANES, LANES)
                    rows_v[base + i, sl] = rows_v[base + i, sl] * w
            return carry2
        lax.fori_loop(0, CHUNK // LANES, grp_body, 0, unroll=False)

        pltpu.sync_copy(rows_v, acc_sh.at[dst_v.at[j]], add=True)
        return carry
    lax.fori_loop(0, NCHUNK, chunk_body, 0, unroll=False)

    plsc.subcore_barrier()
    pltpu.sync_copy(acc_sh.at[pl.ds(row0, ROWS_PER_TILE)],
                    out_hbm.at[c, pl.ds(row0, ROWS_PER_TILE)])

    @pl.when(s == NS - 1)
    def _():
        pltpu.sync_copy(acc_sh.at[pl.ds(NS * ROWS_PER_TILE, ROWS_TAIL)],
                        out_hbm.at[c, pl.ds(NS * ROWS_PER_TILE, ROWS_TAIL)])


_sc_agg = pl.kernel(
    _sc_agg_body,
    out_type=jax.ShapeDtypeStruct((NC, N, D), jnp.float32),
    mesh=plsc.VectorSubcoreMesh(core_axis_name="c", subcore_axis_name="s"),
    scratch_types=[
        pltpu.VMEM((NCHUNK, CHUNK), jnp.int32),
        pltpu.VMEM((NCHUNK, CHUNK), jnp.int32),
        pltpu.VMEM((NCHUNK, CHUNK), jnp.float32),
        pltpu.VMEM((CHUNK, D), jnp.float32),
        pltpu.VMEM_SHARED((N, D), jnp.float32),
        pltpu.SemaphoreType.DMA,
    ],
)


def _mm_body(x_ref, w_ref, o_ref):
    o_ref[...] = jnp.dot(x_ref[...], w_ref[...],
                         preferred_element_type=jnp.float32)


_tc_mm = pl.pallas_call(
    _mm_body,
    out_shape=jax.ShapeDtypeStruct((N, D), jnp.float32),
)


def _fuse_mm_body(p_ref, b_ref, w_ref, o_ref):
    h = jnp.maximum(p_ref[0] + p_ref[1] + b_ref[...], 0.0)
    o_ref[...] = jnp.dot(h, w_ref[...], preferred_element_type=jnp.float32)


_tc_fuse_mm = pl.pallas_call(
    _fuse_mm_body,
    out_shape=jax.ShapeDtypeStruct((N, D), jnp.float32),
)


def _final_body(p_ref, b_ref, x_ref, o_ref):
    o_ref[...] = p_ref[0] + p_ref[1] + b_ref[...] + x_ref[...]


_tc_final = pl.pallas_call(
    _final_body,
    out_shape=jax.ShapeDtypeStruct((N, D), jnp.float32),
)


def kernel(init_h, edge_index, edge_weight, W0, b0, W1, b1, W2, b2):
    pad = E_PAD - E
    pad_i = jnp.zeros((pad,), jnp.int32)
    src = jnp.concatenate([edge_index[0], pad_i]).reshape(NC, NS, NCHUNK, CHUNK)
    dst = jnp.concatenate([edge_index[1], pad_i]).reshape(NC, NS, NCHUNK, CHUNK)
    wgt = jnp.concatenate([edge_weight, jnp.zeros((pad,), jnp.float32)]
                          ).reshape(NC, NS, NCHUNK, CHUNK)
    zeros = jnp.zeros((N, D), jnp.float32)

    support = _tc_mm(init_h, W0)
    p = _sc_agg(support, src, dst, wgt, zeros)
    support = _tc_fuse_mm(p, b0.reshape(1, D), W1)
    p = _sc_agg(support, src, dst, wgt, zeros)
    support = _tc_fuse_mm(p, b1.reshape(1, D), W2)
    p = _sc_agg(support, src, dst, wgt, zeros)
    h = _tc_final(p, b2.reshape(1, D), init_h)
    return (h, init_h)
